# baseline (speedup 1.0000x reference)
import functools

import jax
import jax.numpy as jnp
from jax import lax
from jax.experimental import pallas as pl
from jax.experimental.pallas import tpu as pltpu, tpu_sc as plsc

NS = 16
NC = 2
NW = NC * NS


def _make_sc_agg(width, n_rows_acc, n_chunks, K, dtype, tbl_in_spmem):
  rpt = n_rows_acc // NS
  mesh = plsc.VectorSubcoreMesh(core_axis_name="c", subcore_axis_name="s")

  @functools.partial(
      pl.kernel,
      out_type=jax.ShapeDtypeStruct((NC, n_rows_acc, width), dtype),
      mesh=mesh,
      compiler_params=pltpu.CompilerParams(use_tc_tiling_on_sc=False),
      scratch_types=[
          pltpu.VMEM((n_chunks, K), jnp.int32),
          pltpu.VMEM((n_chunks, K), jnp.int32),
          pltpu.VMEM((2, K, width), dtype),
          pltpu.VMEM_SHARED((n_rows_acc, width), dtype),
          pltpu.VMEM_SHARED((n_rows_acc, width), dtype) if tbl_in_spmem
          else None,
          pltpu.SemaphoreType.DMA,
          pltpu.SemaphoreType.DMA,
      ],
  )
  def sc_agg(tbl_hbm, src_hbm, dst_hbm, zeros_hbm, out_hbm,
             src_v, dst_v, rows_v, acc_sh, tbl_sh, sem_a, sem_b):
    c = lax.axis_index("c")
    s = lax.axis_index("s")
    wid = c * NS + s
    pltpu.sync_copy(zeros_hbm.at[pl.ds(s * rpt, rpt)],
                    acc_sh.at[pl.ds(s * rpt, rpt)])
    if tbl_in_spmem:
      pltpu.sync_copy(tbl_hbm.at[pl.ds(s * rpt, rpt)],
                      tbl_sh.at[pl.ds(s * rpt, rpt)])
    pltpu.sync_copy(src_hbm.at[wid], src_v)
    pltpu.sync_copy(dst_hbm.at[wid], dst_v)
    plsc.subcore_barrier()

    tbl = tbl_sh if tbl_in_spmem else tbl_hbm

    def gather(ci, buf, sem):
      return pltpu.make_async_copy(tbl.at[src_v.at[ci]],
                                   rows_v.at[buf], sem)

    def scatter(ci, buf):
      pltpu.sync_copy(rows_v.at[buf], acc_sh.at[dst_v.at[ci]], add=True)

    gather(0, 0, sem_a).start()

    def body(p, carry):
      ci = 2 * p

      @pl.when(ci + 1 < n_chunks)
      def _():
        gather(ci + 1, 1, sem_b).start()

      gather(ci, 0, sem_a).wait()
      scatter(ci, 0)

      @pl.when(ci + 2 < n_chunks)
      def _():
        gather(ci + 2, 0, sem_a).start()

      @pl.when(ci + 1 < n_chunks)
      def _():
        gather(ci + 1, 1, sem_b).wait()
        scatter(ci + 1, 1)

      return carry

    lax.fori_loop(0, -(-n_chunks // 2), body, 0)
    plsc.subcore_barrier()
    pltpu.sync_copy(acc_sh.at[pl.ds(s * rpt, rpt)],
                    out_hbm.at[c, pl.ds(s * rpt, rpt)])

  return sc_agg


def _tc1_body(pa_ref, x_ref, w1l_ref, b1_ref, w1r_ref, w2l_ref, w2r_ref,
              b2_ref, y_ref, r_ref, inv_ref, *, d):
  agg = (pa_ref[0].astype(jnp.float32)
         + pa_ref[1].astype(jnp.float32))
  cnt = agg[:, d:d + 1]
  inv = 1.0 / jnp.maximum(cnt, 1.0)
  mean = agg[:, :d] * inv
  dims = (((1,), (1,)), ((), ()))
  f32 = jnp.float32
  bf = lambda a: a.astype(jnp.bfloat16)
  z = lax.dot_general(bf(mean), bf(w1l_ref[...]), dims,
                      preferred_element_type=f32)
  z = z + b1_ref[...] + lax.dot_general(bf(x_ref[...]), bf(w1r_ref[...]),
                                        dims, preferred_element_type=f32)
  z = jnp.maximum(z, 0.0)
  zb = bf(z)
  y_ref[...] = lax.dot_general(zb, bf(w2l_ref[...]), dims,
                               preferred_element_type=f32)
  r_ref[...] = lax.dot_general(zb, bf(w2r_ref[...]), dims,
                               preferred_element_type=f32) + b2_ref[...]
  inv_ref[...] = jnp.broadcast_to(inv, r_ref.shape)


def _tc2_body(pb_ref, inv_ref, r_ref, out_ref):
  out_ref[...] = (pb_ref[0] + pb_ref[1]) * inv_ref[...] + r_ref[...]


def kernel(x, edge_index, W1_l, b1, W1_r, W2_l, b2, W2_r):
  n, d = x.shape
  h = W1_l.shape[0]
  out_dim = W2_l.shape[0]
  e = edge_index.shape[1]
  wext = d + 16

  kk = 128
  n_chunks = 8 * (-(-e // (NW * kk * 8)))
  e_pad = NW * kk * n_chunks
  src = jnp.concatenate(
      [edge_index[0], jnp.full((e_pad - e,), n, jnp.int32)]).reshape(
          NW, n_chunks, kk)
  dst = jnp.concatenate(
      [edge_index[1], jnp.full((e_pad - e,), n, jnp.int32)]).reshape(
          NW, n_chunks, kk)

  n_acc = NS * 8 * (-(-(n + 1) // (NS * 8)))

  rpt = n_acc // NS

  xe = jnp.concatenate([x.astype(jnp.bfloat16),
                        jnp.ones((n, NS), jnp.bfloat16)], axis=1)
  xe = jnp.concatenate(
      [xe, jnp.zeros((n_acc - n, wext), jnp.bfloat16)], axis=0)

  sc1 = _make_sc_agg(wext, n_acc, n_chunks, kk, jnp.bfloat16, True)
  pa = sc1(xe, src, dst, jnp.zeros((n_acc, wext), jnp.bfloat16))

  blk = 1000
  grid = n // blk
  full = lambda shape: pl.BlockSpec(shape, lambda i: (0,) * len(shape))
  y, r, inv = pl.pallas_call(
      functools.partial(_tc1_body, d=d),
      grid=(grid,),
      in_specs=[
          pl.BlockSpec((NC, blk, wext), lambda i: (0, i, 0)),
          pl.BlockSpec((blk, d), lambda i: (i, 0)),
          full((h, d)),
          full((1, h)),
          full((h, d)),
          full((out_dim, h)),
          full((out_dim, h)),
          full((1, out_dim)),
      ],
      out_specs=[
          pl.BlockSpec((blk, out_dim), lambda i: (i, 0)),
          pl.BlockSpec((blk, out_dim), lambda i: (i, 0)),
          pl.BlockSpec((blk, out_dim), lambda i: (i, 0)),
      ],
      out_shape=[
          jax.ShapeDtypeStruct((n, out_dim), jnp.float32),
          jax.ShapeDtypeStruct((n, out_dim), jnp.float32),
          jax.ShapeDtypeStruct((n, out_dim), jnp.float32),
      ],
  )(pa, x, W1_l, b1.reshape(1, h), W1_r, W2_l, W2_r, b2.reshape(1, out_dim))

  ye = jnp.concatenate(
      [y, jnp.zeros((n_acc - n, out_dim), jnp.float32)], axis=0)
  sc2 = _make_sc_agg(out_dim, n_acc, n_chunks, kk, jnp.float32, True)
  pb = sc2(ye, src, dst, jnp.zeros((n_acc, out_dim), jnp.float32))

  out = pl.pallas_call(
      _tc2_body,
      grid=(grid,),
      in_specs=[
          pl.BlockSpec((NC, blk, out_dim), lambda i: (0, i, 0)),
          pl.BlockSpec((blk, out_dim), lambda i: (i, 0)),
          pl.BlockSpec((blk, out_dim), lambda i: (i, 0)),
      ],
      out_specs=pl.BlockSpec((blk, out_dim), lambda i: (i, 0)),
      out_shape=jax.ShapeDtypeStruct((n, out_dim), jnp.float32),
  )(pb, inv, r)
  return out


# speedup vs baseline: 1.0401x; 1.0401x over previous
import functools

import jax
import jax.numpy as jnp
from jax import lax
from jax.experimental import pallas as pl
from jax.experimental.pallas import tpu as pltpu, tpu_sc as plsc

NS = 16
NC = 2
NW = NC * NS
KK = 128
CW = 16


def _make_sc_agg(n_rows_acc, n_chunks, with_cnt):
  rpt = n_rows_acc // NS
  mesh = plsc.VectorSubcoreMesh(core_axis_name="c", subcore_axis_name="s")
  bf = jnp.bfloat16

  out_type = [jax.ShapeDtypeStruct((NC, n_rows_acc, 128), bf)]
  scratch = [
      pltpu.VMEM((n_chunks, KK), jnp.int32),
      pltpu.VMEM((n_chunks, KK), jnp.int32),
      pltpu.VMEM((2, KK, 128), bf),
      pltpu.VMEM_SHARED((n_rows_acc, 128), bf),
      pltpu.VMEM_SHARED((n_rows_acc, 128), bf),
      pltpu.SemaphoreType.DMA,
      pltpu.SemaphoreType.DMA,
  ]
  if with_cnt:
    out_type.append(jax.ShapeDtypeStruct((NC, n_rows_acc, CW), bf))
    scratch += [pltpu.VMEM((KK, CW), bf),
                pltpu.VMEM_SHARED((n_rows_acc, CW), bf)]

  @functools.partial(
      pl.kernel,
      out_type=out_type,
      mesh=mesh,
      compiler_params=pltpu.CompilerParams(use_tc_tiling_on_sc=False),
      scratch_types=scratch,
  )
  def sc_agg(tbl_hbm, edges_hbm, zeros_hbm, ones_hbm, *refs):
    if with_cnt:
      (out_hbm, cnt_out_hbm, src_v, dst_v, rows_v, acc_sh, tbl_sh,
       sem_a, sem_b, ones_v, cnt_sh) = refs
    else:
      (out_hbm, src_v, dst_v, rows_v, acc_sh, tbl_sh, sem_a, sem_b) = refs
    c = lax.axis_index("c")
    s = lax.axis_index("s")
    wid = c * NS + s
    row0 = s * rpt
    pltpu.sync_copy(zeros_hbm.at[pl.ds(row0, rpt)],
                    acc_sh.at[pl.ds(row0, rpt)])
    pltpu.sync_copy(tbl_hbm.at[pl.ds(row0, rpt)],
                    tbl_sh.at[pl.ds(row0, rpt)])
    pltpu.sync_copy(edges_hbm.at[0, pl.ds(wid * n_chunks, n_chunks)], src_v)
    pltpu.sync_copy(edges_hbm.at[1, pl.ds(wid * n_chunks, n_chunks)], dst_v)
    if with_cnt:
      pltpu.sync_copy(zeros_hbm.at[pl.ds(row0, rpt), pl.ds(0, CW)],
                      cnt_sh.at[pl.ds(row0, rpt)])
      pltpu.sync_copy(ones_hbm, ones_v)
    plsc.subcore_barrier()

    def gather(ci, buf, sem):
      return pltpu.make_async_copy(tbl_sh.at[src_v.at[ci]],
                                   rows_v.at[buf], sem)

    def scatter(ci, buf):
      pltpu.sync_copy(rows_v.at[buf], acc_sh.at[dst_v.at[ci]], add=True)
      if with_cnt:
        pltpu.sync_copy(ones_v, cnt_sh.at[dst_v.at[ci]], add=True)

    gather(0, 0, sem_a).start()

    def body(p, carry):
      ci = 2 * p

      @pl.when(ci + 1 < n_chunks)
      def _():
        gather(ci + 1, 1, sem_b).start()

      gather(ci, 0, sem_a).wait()
      scatter(ci, 0)

      @pl.when(ci + 2 < n_chunks)
      def _():
        gather(ci + 2, 0, sem_a).start()

      @pl.when(ci + 1 < n_chunks)
      def _():
        gather(ci + 1, 1, sem_b).wait()
        scatter(ci + 1, 1)

      return carry

    lax.fori_loop(0, -(-n_chunks // 2), body, 0)
    plsc.subcore_barrier()
    pltpu.sync_copy(acc_sh.at[pl.ds(row0, rpt)],
                    out_hbm.at[c, pl.ds(row0, rpt)])
    if with_cnt:
      pltpu.sync_copy(cnt_sh.at[pl.ds(row0, rpt)],
                      cnt_out_hbm.at[c, pl.ds(row0, rpt)])

  return sc_agg


def _tc1_body(pa_ref, x_ref, inv_ref, w1l_ref, b1_ref, w1r_ref, w2l_ref,
              w2r_ref, b2_ref, y_ref, r_ref, *, n, blk):
  agg = (pa_ref[0] + pa_ref[1]).astype(jnp.float32)
  inv = inv_ref[...]
  mean = agg * inv
  dims = (((1,), (1,)), ((), ()))
  f32 = jnp.float32
  bf = lambda a: a.astype(jnp.bfloat16)
  z = lax.dot_general(bf(mean), bf(w1l_ref[...]), dims,
                      preferred_element_type=f32)
  z = z + b1_ref[...] + lax.dot_general(bf(x_ref[...]), bf(w1r_ref[...]),
                                        dims, preferred_element_type=f32)
  z = jnp.maximum(z, 0.0)
  zb = bf(z)
  y = lax.dot_general(zb, bf(w2l_ref[...]), dims, preferred_element_type=f32)
  rows = pl.program_id(0) * blk + lax.broadcasted_iota(jnp.int32, (blk, 1), 0)
  y = jnp.where(rows < n, y, 0.0)
  y_ref[...] = jnp.concatenate(
      [y.astype(jnp.bfloat16),
       jnp.zeros((blk, 128 - y.shape[1]), jnp.bfloat16)], axis=1)
  r_ref[...] = lax.dot_general(zb, bf(w2r_ref[...]), dims,
                               preferred_element_type=f32) + b2_ref[...]


def _tc2_body(pb_ref, inv_ref, r_ref, out_ref, *, out_dim, blk):
  agg = (pb_ref[0, :, :out_dim] + pb_ref[1, :, :out_dim]).astype(jnp.float32)
  out_ref[...] = agg * inv_ref[...] + r_ref[...]


def kernel(x, edge_index, W1_l, b1, W1_r, W2_l, b2, W2_r):
  n, d = x.shape
  h = W1_l.shape[0]
  out_dim = W2_l.shape[0]
  e = edge_index.shape[1]

  n_chunks = 8 * (-(-e // (NW * KK * 8)))
  e_pad = NW * KK * n_chunks
  edges = jnp.concatenate(
      [edge_index, jnp.full((2, e_pad - e), n, jnp.int32)],
      axis=1).reshape(2, NW * n_chunks, KK)

  n_acc = NS * 8 * (-(-(n + 1) // (NS * 8)))

  bfl = jnp.bfloat16
  tbl1 = jnp.pad(x.astype(bfl), ((0, n_acc - n), (0, 0)))
  zeros = jnp.zeros((n_acc, 128), bfl)
  ones_in = jnp.ones((KK, CW), bfl)

  sc1 = _make_sc_agg(n_acc, n_chunks, True)
  pa, pcnt = sc1(tbl1, edges, zeros, ones_in)

  cnt = (pcnt[0, :, :1] + pcnt[1, :, :1]).astype(jnp.float32)
  inv = 1.0 / jnp.maximum(cnt, 1.0)

  xp = jnp.pad(x, ((0, n_acc - n), (0, 0)))

  blk1 = n_acc // 8
  full = lambda shape: pl.BlockSpec(shape, lambda i: (0,) * len(shape))
  y, r = pl.pallas_call(
      functools.partial(_tc1_body, n=n, blk=blk1),
      grid=(8,),
      in_specs=[
          pl.BlockSpec((NC, blk1, 128), lambda i: (0, i, 0)),
          pl.BlockSpec((blk1, d), lambda i: (i, 0)),
          pl.BlockSpec((blk1, 1), lambda i: (i, 0)),
          full((h, d)),
          full((1, h)),
          full((h, d)),
          full((out_dim, h)),
          full((out_dim, h)),
          full((1, out_dim)),
      ],
      out_specs=[
          pl.BlockSpec((blk1, 128), lambda i: (i, 0)),
          pl.BlockSpec((blk1, out_dim), lambda i: (i, 0)),
      ],
      out_shape=[
          jax.ShapeDtypeStruct((n_acc, 128), bfl),
          jax.ShapeDtypeStruct((n_acc, out_dim), jnp.float32),
      ],
  )(pa, xp, inv, W1_l, b1.reshape(1, h), W1_r, W2_l, W2_r,
    b2.reshape(1, out_dim))

  sc2 = _make_sc_agg(n_acc, n_chunks, False)
  (pb,) = sc2(y, edges, zeros, ones_in)

  blk2 = n // 5
  out = pl.pallas_call(
      functools.partial(_tc2_body, out_dim=out_dim, blk=blk2),
      grid=(5,),
      in_specs=[
          pl.BlockSpec((NC, blk2, 128), lambda i: (0, i, 0)),
          pl.BlockSpec((blk2, 1), lambda i: (i, 0)),
          pl.BlockSpec((blk2, out_dim), lambda i: (i, 0)),
      ],
      out_specs=pl.BlockSpec((blk2, out_dim), lambda i: (i, 0)),
      out_shape=jax.ShapeDtypeStruct((n, out_dim), jnp.float32),
  )(pb, inv, r)
  return out


# speedup vs baseline: 1.2412x; 1.1934x over previous
import functools

import jax
import jax.numpy as jnp
from jax import lax
from jax.experimental import pallas as pl
from jax.experimental.pallas import tpu as pltpu, tpu_sc as plsc

NS = 16
NC = 2
NW = NC * NS
KK = 128
CW = 16


def _make_sc_agg(n_rows_acc, n_chunks, width, with_cnt):
  rpt = n_rows_acc // NS
  mesh = plsc.VectorSubcoreMesh(core_axis_name="c", subcore_axis_name="s")
  bf = jnp.bfloat16

  out_type = [jax.ShapeDtypeStruct((NC, n_rows_acc, width), bf)]
  scratch = [
      pltpu.VMEM((n_chunks, KK), jnp.int32),
      pltpu.VMEM((n_chunks, KK), jnp.int32),
      pltpu.VMEM((2, KK, width), bf),
      pltpu.VMEM_SHARED((n_rows_acc, width), bf),
      pltpu.VMEM_SHARED((n_rows_acc, width), bf),
      pltpu.SemaphoreType.DMA,
      pltpu.SemaphoreType.DMA,
  ]
  if with_cnt:
    out_type.append(jax.ShapeDtypeStruct((NC, n_rows_acc, CW), bf))
    scratch += [pltpu.VMEM((KK, CW), bf),
                pltpu.VMEM_SHARED((n_rows_acc, CW), bf)]

  @functools.partial(
      pl.kernel,
      out_type=out_type,
      mesh=mesh,
      compiler_params=pltpu.CompilerParams(use_tc_tiling_on_sc=False),
      scratch_types=scratch,
  )
  def sc_agg(tbl_hbm, edges_hbm, zeros_hbm, ones_hbm, *refs):
    if with_cnt:
      (out_hbm, cnt_out_hbm, src_v, dst_v, rows_v, acc_sh, tbl_sh,
       sem_a, sem_b, ones_v, cnt_sh) = refs
    else:
      (out_hbm, src_v, dst_v, rows_v, acc_sh, tbl_sh, sem_a, sem_b) = refs
    c = lax.axis_index("c")
    s = lax.axis_index("s")
    wid = c * NS + s
    row0 = s * rpt
    pltpu.sync_copy(zeros_hbm.at[pl.ds(row0, rpt), pl.ds(0, width)],
                    acc_sh.at[pl.ds(row0, rpt)])
    pltpu.sync_copy(tbl_hbm.at[pl.ds(row0, rpt)],
                    tbl_sh.at[pl.ds(row0, rpt)])
    pltpu.sync_copy(edges_hbm.at[0, pl.ds(wid * n_chunks, n_chunks)], src_v)
    pltpu.sync_copy(edges_hbm.at[1, pl.ds(wid * n_chunks, n_chunks)], dst_v)
    if with_cnt:
      pltpu.sync_copy(ones_hbm, ones_v)
      pltpu.sync_copy(zeros_hbm.at[pl.ds(row0, rpt), pl.ds(0, CW)],
                      cnt_sh.at[pl.ds(row0, rpt)])
    plsc.subcore_barrier()

    def gather(ci, buf, sem):
      return pltpu.make_async_copy(tbl_sh.at[src_v.at[ci]],
                                   rows_v.at[buf], sem)

    def scatter(ci, buf):
      pltpu.sync_copy(rows_v.at[buf], acc_sh.at[dst_v.at[ci]], add=True)
      if with_cnt:
        pltpu.sync_copy(ones_v, cnt_sh.at[dst_v.at[ci]], add=True)

    gather(0, 0, sem_a).start()

    def body(p, carry):
      ci = 2 * p

      @pl.when(ci + 1 < n_chunks)
      def _():
        gather(ci + 1, 1, sem_b).start()

      gather(ci, 0, sem_a).wait()
      scatter(ci, 0)

      @pl.when(ci + 2 < n_chunks)
      def _():
        gather(ci + 2, 0, sem_a).start()

      @pl.when(ci + 1 < n_chunks)
      def _():
        gather(ci + 1, 1, sem_b).wait()
        scatter(ci + 1, 1)

      return carry

    lax.fori_loop(0, -(-n_chunks // 2), body, 0)
    plsc.subcore_barrier()
    pltpu.sync_copy(acc_sh.at[pl.ds(row0, rpt)],
                    out_hbm.at[c, pl.ds(row0, rpt)])
    if with_cnt:
      pltpu.sync_copy(cnt_sh.at[pl.ds(row0, rpt)],
                      cnt_out_hbm.at[c, pl.ds(row0, rpt)])

  return sc_agg


def _tc1_body(pa_ref, x_ref, inv_ref, w1l_ref, b1_ref, w1r_ref, w2l_ref,
              w2r_ref, b2_ref, y_ref, r_ref, *, n, blk):
  agg = (pa_ref[0] + pa_ref[1]).astype(jnp.float32)
  inv = inv_ref[...]
  mean = agg * inv
  dims = (((1,), (1,)), ((), ()))
  f32 = jnp.float32
  bf = lambda a: a.astype(jnp.bfloat16)
  z = lax.dot_general(bf(mean), bf(w1l_ref[...]), dims,
                      preferred_element_type=f32)
  z = z + b1_ref[...] + lax.dot_general(bf(x_ref[...]), bf(w1r_ref[...]),
                                        dims, preferred_element_type=f32)
  z = jnp.maximum(z, 0.0)
  zb = bf(z)
  y = lax.dot_general(zb, bf(w2l_ref[...]), dims, preferred_element_type=f32)
  rows = pl.program_id(0) * blk + lax.broadcasted_iota(jnp.int32, (blk, 1), 0)
  y = jnp.where(rows < n, y, 0.0)
  y_ref[...] = y.astype(jnp.bfloat16)
  r_ref[...] = lax.dot_general(zb, bf(w2r_ref[...]), dims,
                               preferred_element_type=f32) + b2_ref[...]


def _tc2_body(pb_ref, inv_ref, r_ref, out_ref, *, out_dim, blk):
  agg = (pb_ref[0] + pb_ref[1]).astype(jnp.float32)
  out_ref[...] = agg * inv_ref[...] + r_ref[...]


def kernel(x, edge_index, W1_l, b1, W1_r, W2_l, b2, W2_r):
  n, d = x.shape
  h = W1_l.shape[0]
  out_dim = W2_l.shape[0]
  e = edge_index.shape[1]

  n_chunks = 8 * (-(-e // (NW * KK * 8)))
  e_pad = NW * KK * n_chunks
  edges = jnp.concatenate(
      [edge_index, jnp.full((2, e_pad - e), n, jnp.int32)],
      axis=1).reshape(2, NW * n_chunks, KK)

  n_acc = NS * 8 * (-(-(n + 1) // (NS * 8)))

  bfl = jnp.bfloat16
  tbl1 = jnp.pad(x.astype(bfl), ((0, n_acc - n), (0, 0)))
  zeros = jnp.zeros((n_acc, 128), bfl)
  ones_in = jnp.ones((KK, CW), bfl)

  sc1 = _make_sc_agg(n_acc, n_chunks, 128, True)
  pa, pcnt = sc1(tbl1, edges, zeros, ones_in)

  cnt = (pcnt[0, :, :1] + pcnt[1, :, :1]).astype(jnp.float32)
  inv = 1.0 / jnp.maximum(cnt, 1.0)

  xp = jnp.pad(x, ((0, n_acc - n), (0, 0)))

  blk1 = n_acc // 8
  full = lambda shape: pl.BlockSpec(shape, lambda i: (0,) * len(shape))
  y, r = pl.pallas_call(
      functools.partial(_tc1_body, n=n, blk=blk1),
      grid=(8,),
      in_specs=[
          pl.BlockSpec((NC, blk1, 128), lambda i: (0, i, 0)),
          pl.BlockSpec((blk1, d), lambda i: (i, 0)),
          pl.BlockSpec((blk1, 1), lambda i: (i, 0)),
          full((h, d)),
          full((1, h)),
          full((h, d)),
          full((out_dim, h)),
          full((out_dim, h)),
          full((1, out_dim)),
      ],
      out_specs=[
          pl.BlockSpec((blk1, out_dim), lambda i: (i, 0)),
          pl.BlockSpec((blk1, out_dim), lambda i: (i, 0)),
      ],
      out_shape=[
          jax.ShapeDtypeStruct((n_acc, out_dim), bfl),
          jax.ShapeDtypeStruct((n_acc, out_dim), jnp.float32),
      ],
  )(pa, xp, inv, W1_l, b1.reshape(1, h), W1_r, W2_l, W2_r,
    b2.reshape(1, out_dim))

  sc2 = _make_sc_agg(n_acc, n_chunks, out_dim, False)
  (pb,) = sc2(y, edges, zeros, ones_in)

  blk2 = n // 5
  out = pl.pallas_call(
      functools.partial(_tc2_body, out_dim=out_dim, blk=blk2),
      grid=(5,),
      in_specs=[
          pl.BlockSpec((NC, blk2, out_dim), lambda i: (0, i, 0)),
          pl.BlockSpec((blk2, 1), lambda i: (i, 0)),
          pl.BlockSpec((blk2, out_dim), lambda i: (i, 0)),
      ],
      out_specs=pl.BlockSpec((blk2, out_dim), lambda i: (i, 0)),
      out_shape=jax.ShapeDtypeStruct((n, out_dim), jnp.float32),
  )(pb, inv, r)
  return out
